# baseline (speedup 1.0000x reference)
import functools

import jax
import jax.numpy as jnp
import numpy as np
from jax import lax
from jax.experimental import pallas as pl
from jax.experimental.pallas import tpu as pltpu
from jax.experimental.pallas import tpu_sc as plsc

B, L, D, P = 64, 32, 4096, 64
K = 512
DBLK = 4096
BBLK = 16

NC, NS, LANES = 2, 16, 16
NW = NC * NS
ROWS_PER_W = B // NW
NCHUNK = D // LANES
MSB = np.int32(-2147483648)


def _scores_body(v_ref, x_ref, s_ref):
    acc = x_ref[:, 0, :] * v_ref[0]
    for l in range(1, L):
        acc += x_ref[:, l, :] * v_ref[l]
    s_ref[...] = acc


UNR = 1


def _scan_hist(hist_v, kcnt, lane_iota):
    cum = jnp.float32(0.0)
    dvec = jnp.zeros((LANES,), jnp.float32)
    svec = jnp.zeros((LANES,), jnp.float32)
    for c in range(15, -1, -1):
        v = hist_v[pl.ds(c * LANES, LANES)]
        pc = plsc.cumsum(v)
        stot = jnp.sum(v)
        cntgt = cum + (stot - pc)
        cond = (cntgt < kcnt) & (cntgt + v >= kcnt)
        dvec += jnp.where(cond, lane_iota + (c * 16), 0.0)
        svec += jnp.where(cond, cntgt, 0.0)
        cum += stot
    return jnp.sum(dvec).astype(jnp.int32), jnp.sum(svec)


def _sc_topk_body(scores_hbm, w_hbm, z_hbm, row0_v, row1_v, key0_v, key1_v,
                  w0_v, w1_v, hist0_v, hist1_v, z0_v, z1_v, sem0, sem1):
    wid = lax.axis_index("s") * NC + lax.axis_index("c")
    r0 = wid * ROWS_PER_W
    cin0 = pltpu.make_async_copy(scores_hbm.at[r0], row0_v, sem0)
    cin1 = pltpu.make_async_copy(scores_hbm.at[r0 + 1], row1_v, sem1)
    cin0.start()
    cin1.start()
    cin0.wait()
    cin1.wait()

    ones = jnp.ones((LANES,), jnp.float32)
    zeros16 = jnp.zeros((LANES,), jnp.float32)
    lane_iota = lax.iota(jnp.int32, LANES).astype(jnp.float32)
    rows = ((row0_v, key0_v, w0_v, hist0_v),
            (row1_v, key1_v, w1_v, hist1_v))

    def zero_hists():
        for c in range(16):
            hist0_v[pl.ds(c * LANES, LANES)] = zeros16
            hist1_v[pl.ds(c * LANES, LANES)] = zeros16

    zero_hists()

    @plsc.parallel_loop(0, NCHUNK // UNR, 1, unroll=8,
                        carry=(jnp.full((LANES,), -3.4e38, jnp.float32),
                               jnp.full((LANES,), -3.4e38, jnp.float32)))
    def pass1(i, carry):
        mxs = list(carry)
        for u in range(UNR):
            off = (i * UNR + u) * LANES
            for ri, (rv, kv, _, hv) in enumerate(rows):
                s = rv[pl.ds(off, LANES)]
                bits = lax.bitcast_convert_type(s, jnp.int32)
                uk = (bits ^ (lax.shift_right_arithmetic(bits, 31)
                              & jnp.int32(0x7FFFFFFF))) ^ MSB
                kv[pl.ds(off, LANES)] = uk
                plsc.addupdate_scatter(hv, [lax.shift_right_logical(uk, 24)],
                                       ones)
                mxs[ri] = jnp.maximum(mxs[ri], s)
        return tuple(mxs)

    mx0, mx1 = pass1
    rmaxs = [jnp.max(mx0), jnp.max(mx1)]

    kcnts = [jnp.float32(K), jnp.float32(K)]
    prefs = [jnp.int32(0), jnp.int32(0)]
    for ri in range(2):
        d, sub = _scan_hist(rows[ri][3], kcnts[ri], lane_iota)
        kcnts[ri] -= sub
        prefs[ri] = d

    for rnd in range(1, 4):
        shift = 24 - 8 * rnd
        zero_hists()

        @plsc.parallel_loop(0, NCHUNK // UNR, 1, unroll=8)
        def dpass(i, shift=shift, prefs=prefs):
            for u in range(UNR):
                off = (i * UNR + u) * LANES
                for ri, (_, kv, _, hv) in enumerate(rows):
                    uk = kv[pl.ds(off, LANES)]
                    m = lax.shift_right_logical(uk, shift + 8) == prefs[ri]
                    dig = lax.shift_right_logical(uk, shift) & jnp.int32(0xFF)
                    plsc.addupdate_scatter(hv, [dig], ones, mask=m)
        for ri in range(2):
            d, sub = _scan_hist(rows[ri][3], kcnts[ri], lane_iota)
            kcnts[ri] -= sub
            prefs[ri] = prefs[ri] * 256 + d

    tkeys = [prefs[0] ^ MSB, prefs[1] ^ MSB]

    @plsc.parallel_loop(0, NCHUNK // UNR, 1, unroll=8,
                        carry=(zeros16, zeros16))
    def fpass(i, carry):
        accs = list(carry)
        for u in range(UNR):
            off = (i * UNR + u) * LANES
            for ri, (rv, kv, wv, _) in enumerate(rows):
                s = rv[pl.ds(off, LANES)]
                uk = kv[pl.ds(off, LANES)]
                m = (uk ^ MSB) >= tkeys[ri]
                w = jnp.where(m, jnp.exp(s - rmaxs[ri]), 0.0)
                wv[pl.ds(off, LANES)] = w
                accs[ri] = accs[ri] + w
        return tuple(accs)

    acc0, acc1 = fpass
    z0_v[...] = jnp.sum(acc0) * ones
    z1_v[...] = jnp.sum(acc1) * ones
    cout0 = pltpu.make_async_copy(w0_v, w_hbm.at[r0], sem0)
    cout1 = pltpu.make_async_copy(w1_v, w_hbm.at[r0 + 1], sem1)
    cout0.start()
    cout1.start()
    pltpu.sync_copy(z0_v, z_hbm.at[r0])
    pltpu.sync_copy(z1_v, z_hbm.at[r0 + 1])
    cout0.wait()
    cout1.wait()


_sc_topk = functools.partial(
    pl.kernel,
    mesh=plsc.VectorSubcoreMesh(core_axis_name="c", subcore_axis_name="s"),
    compiler_params=pltpu.CompilerParams(needs_layout_passes=False),
    out_type=[
        jax.ShapeDtypeStruct((B, D), jnp.float32),
        jax.ShapeDtypeStruct((B, LANES), jnp.float32),
    ],
    scratch_types=[
        pltpu.VMEM((D,), jnp.float32),
        pltpu.VMEM((D,), jnp.float32),
        pltpu.VMEM((D,), jnp.int32),
        pltpu.VMEM((D,), jnp.int32),
        pltpu.VMEM((D,), jnp.float32),
        pltpu.VMEM((D,), jnp.float32),
        pltpu.VMEM((256,), jnp.float32),
        pltpu.VMEM((256,), jnp.float32),
        pltpu.VMEM((LANES,), jnp.float32),
        pltpu.VMEM((LANES,), jnp.float32),
        pltpu.SemaphoreType.DMA,
        pltpu.SemaphoreType.DMA,
    ],
)(_sc_topk_body)


def _weighted_reduce_body(x_ref, wd_ref, r_ref):
    j = pl.program_id(1)

    @pl.when(j == 0)
    def _():
        r_ref[...] = jnp.zeros_like(r_ref)

    r_ref[...] += jnp.sum(x_ref[...] * wd_ref[...][:, None, :], axis=2)


def _head_body(r_ref, z_ref, w1_ref, b1_ref, g_ref, bb_ref, wh1_ref, bh1_ref,
               wh2_ref, bh2_ref, out_ref):
    r = r_ref[...] / z_ref[...][:, 0:1]
    out = lax.dot_general(r, w1_ref[...], (((1,), (1,)), ((), ())),
                          precision=lax.Precision.HIGHEST,
                          preferred_element_type=jnp.float32) + b1_ref[...]
    mu = jnp.mean(out, axis=1, keepdims=True)
    dlt = out - mu
    var = jnp.mean(dlt * dlt, axis=1, keepdims=True)
    outn = dlt * lax.rsqrt(var + 1e-5) * g_ref[...] + bb_ref[...]
    h1 = lax.dot_general(outn, wh1_ref[...], (((1,), (1,)), ((), ())),
                         precision=lax.Precision.HIGHEST,
                         preferred_element_type=jnp.float32) + bh1_ref[...]
    h1 = 0.5 * h1 * (1.0 + lax.erf(h1 * (2.0 ** -0.5)))
    out_ref[...] = lax.dot_general(h1, wh2_ref[...], (((1,), (1,)), ((), ())),
                                   precision=lax.Precision.HIGHEST,
                                   preferred_element_type=jnp.float32) + bh2_ref[...]


@jax.jit
def kernel(x, W1, b1, Ws, bs, ln_g, ln_b, Wh1, bh1, Wh2, bh2):
    v = jnp.einsum("pl,p->l", W1, Ws[0])

    scores = pl.pallas_call(
        _scores_body,
        grid=(B // BBLK, D // DBLK),
        in_specs=[
            pl.BlockSpec(memory_space=pltpu.SMEM),
            pl.BlockSpec((BBLK, L, DBLK), lambda i, j: (i, 0, j)),
        ],
        out_specs=pl.BlockSpec((BBLK, DBLK), lambda i, j: (i, j)),
        out_shape=jax.ShapeDtypeStruct((B, D), jnp.float32),
    )(v, x)

    w_field, z = _sc_topk(scores)

    r = pl.pallas_call(
        _weighted_reduce_body,
        grid=(B // BBLK, D // DBLK),
        in_specs=[
            pl.BlockSpec((BBLK, L, DBLK), lambda i, j: (i, 0, j)),
            pl.BlockSpec((BBLK, DBLK), lambda i, j: (i, j)),
        ],
        out_specs=pl.BlockSpec((BBLK, L), lambda i, j: (i, 0)),
        out_shape=jax.ShapeDtypeStruct((B, L), jnp.float32),
    )(x, w_field)

    logits = pl.pallas_call(
        _head_body,
        out_shape=jax.ShapeDtypeStruct((B, 2), jnp.float32),
    )(r, z, W1, b1.reshape(1, P), ln_g.reshape(1, P), ln_b.reshape(1, P),
      Wh1, bh1.reshape(1, 128), Wh2, bh2.reshape(1, 2))
    return logits


# speedup vs baseline: 1.0403x; 1.0403x over previous
import functools

import jax
import jax.numpy as jnp
import numpy as np
from jax import lax
from jax.experimental import pallas as pl
from jax.experimental.pallas import tpu as pltpu
from jax.experimental.pallas import tpu_sc as plsc

B, L, D, P = 64, 32, 4096, 64
K = 512
DBLK = 4096
BBLK = 16

NC, NS, LANES = 2, 16, 16
NW = NC * NS
ROWS_PER_W = B // NW
NCHUNK = D // LANES
MSB = np.int32(-2147483648)


def _scores_body(v_ref, x_ref, s_ref):
    acc = x_ref[:, 0, :] * v_ref[0]
    for l in range(1, L):
        acc += x_ref[:, l, :] * v_ref[l]
    s_ref[...] = acc


UNR = 1


def _scan_hist(hist_v, kcnt, lane_iota):
    cum = jnp.float32(0.0)
    dvec = jnp.zeros((LANES,), jnp.float32)
    svec = jnp.zeros((LANES,), jnp.float32)
    for c in range(15, -1, -1):
        v = hist_v[pl.ds(c * LANES, LANES)]
        pc = plsc.cumsum(v)
        stot = jnp.sum(v)
        cntgt = cum + (stot - pc)
        cond = (cntgt < kcnt) & (cntgt + v >= kcnt)
        dvec += jnp.where(cond, lane_iota + (c * 16), 0.0)
        svec += jnp.where(cond, cntgt, 0.0)
        cum += stot
    return jnp.sum(dvec).astype(jnp.int32), jnp.sum(svec)


def _sc_topk_body(scores_hbm, w_hbm, z_hbm, row0_v, row1_v, key0_v, key1_v,
                  w0_v, w1_v, hist0_v, hist1_v, z0_v, z1_v, sem0, sem1):
    wid = lax.axis_index("s") * NC + lax.axis_index("c")
    r0 = wid * ROWS_PER_W
    cin0 = pltpu.make_async_copy(scores_hbm.at[r0], row0_v, sem0)
    cin1 = pltpu.make_async_copy(scores_hbm.at[r0 + 1], row1_v, sem1)
    cin0.start()
    cin1.start()
    cin0.wait()
    cin1.wait()

    ones = jnp.ones((LANES,), jnp.float32)
    zeros16 = jnp.zeros((LANES,), jnp.float32)
    lane_iota = lax.iota(jnp.int32, LANES).astype(jnp.float32)
    rows = ((row0_v, key0_v, w0_v, hist0_v),
            (row1_v, key1_v, w1_v, hist1_v))

    def zero_hists():
        for c in range(16):
            hist0_v[pl.ds(c * LANES, LANES)] = zeros16
            hist1_v[pl.ds(c * LANES, LANES)] = zeros16

    zero_hists()

    @plsc.parallel_loop(0, NCHUNK // UNR, 1, unroll=4,
                        carry=(jnp.full((LANES,), -3.4e38, jnp.float32),
                               jnp.full((LANES,), -3.4e38, jnp.float32)))
    def pass1(i, carry):
        mxs = list(carry)
        for u in range(UNR):
            off = (i * UNR + u) * LANES
            for ri, (rv, kv, _, hv) in enumerate(rows):
                s = rv[pl.ds(off, LANES)]
                bits = lax.bitcast_convert_type(s, jnp.int32)
                uk = (bits ^ (lax.shift_right_arithmetic(bits, 31)
                              & jnp.int32(0x7FFFFFFF))) ^ MSB
                kv[pl.ds(off, LANES)] = uk
                plsc.addupdate_scatter(hv, [lax.shift_right_logical(uk, 24)],
                                       ones)
                mxs[ri] = jnp.maximum(mxs[ri], s)
        return tuple(mxs)

    mx0, mx1 = pass1
    rmaxs = [jnp.max(mx0), jnp.max(mx1)]

    kcnts = [jnp.float32(K), jnp.float32(K)]
    prefs = [jnp.int32(0), jnp.int32(0)]
    for ri in range(2):
        d, sub = _scan_hist(rows[ri][3], kcnts[ri], lane_iota)
        kcnts[ri] -= sub
        prefs[ri] = d

    for rnd in range(1, 4):
        shift = 24 - 8 * rnd
        zero_hists()

        @plsc.parallel_loop(0, NCHUNK // UNR, 1, unroll=4)
        def dpass(i, shift=shift, prefs=prefs):
            for u in range(UNR):
                off = (i * UNR + u) * LANES
                for ri, (_, kv, _, hv) in enumerate(rows):
                    uk = kv[pl.ds(off, LANES)]
                    m = lax.shift_right_logical(uk, shift + 8) == prefs[ri]
                    dig = lax.shift_right_logical(uk, shift) & jnp.int32(0xFF)
                    plsc.addupdate_scatter(hv, [dig], ones, mask=m)
        for ri in range(2):
            d, sub = _scan_hist(rows[ri][3], kcnts[ri], lane_iota)
            kcnts[ri] -= sub
            prefs[ri] = prefs[ri] * 256 + d

    tkeys = [prefs[0] ^ MSB, prefs[1] ^ MSB]

    @plsc.parallel_loop(0, NCHUNK // UNR, 1, unroll=4,
                        carry=(zeros16, zeros16))
    def fpass(i, carry):
        accs = list(carry)
        for u in range(UNR):
            off = (i * UNR + u) * LANES
            for ri, (rv, kv, wv, _) in enumerate(rows):
                s = rv[pl.ds(off, LANES)]
                uk = kv[pl.ds(off, LANES)]
                m = (uk ^ MSB) >= tkeys[ri]
                w = jnp.where(m, jnp.exp(s - rmaxs[ri]), 0.0)
                wv[pl.ds(off, LANES)] = w
                accs[ri] = accs[ri] + w
        return tuple(accs)

    acc0, acc1 = fpass
    z0_v[...] = jnp.sum(acc0) * ones
    z1_v[...] = jnp.sum(acc1) * ones
    cout0 = pltpu.make_async_copy(w0_v, w_hbm.at[r0], sem0)
    cout1 = pltpu.make_async_copy(w1_v, w_hbm.at[r0 + 1], sem1)
    cout0.start()
    cout1.start()
    pltpu.sync_copy(z0_v, z_hbm.at[r0])
    pltpu.sync_copy(z1_v, z_hbm.at[r0 + 1])
    cout0.wait()
    cout1.wait()


_sc_topk = functools.partial(
    pl.kernel,
    mesh=plsc.VectorSubcoreMesh(core_axis_name="c", subcore_axis_name="s"),
    compiler_params=pltpu.CompilerParams(needs_layout_passes=False),
    out_type=[
        jax.ShapeDtypeStruct((B, D), jnp.float32),
        jax.ShapeDtypeStruct((B, LANES), jnp.float32),
    ],
    scratch_types=[
        pltpu.VMEM((D,), jnp.float32),
        pltpu.VMEM((D,), jnp.float32),
        pltpu.VMEM((D,), jnp.int32),
        pltpu.VMEM((D,), jnp.int32),
        pltpu.VMEM((D,), jnp.float32),
        pltpu.VMEM((D,), jnp.float32),
        pltpu.VMEM((256,), jnp.float32),
        pltpu.VMEM((256,), jnp.float32),
        pltpu.VMEM((LANES,), jnp.float32),
        pltpu.VMEM((LANES,), jnp.float32),
        pltpu.SemaphoreType.DMA,
        pltpu.SemaphoreType.DMA,
    ],
)(_sc_topk_body)


DBLK3 = 1024
NSTEP3 = D // DBLK3


def _reduce_head_body(x_ref, wd_ref, z_ref, w1_ref, b1_ref, g_ref, bb_ref,
                      wh1_ref, bh1_ref, wh2_ref, bh2_ref, out_ref, racc_ref):
    j = pl.program_id(0)

    @pl.when(j == 0)
    def _():
        racc_ref[...] = jnp.zeros_like(racc_ref)

    racc_ref[...] += jnp.sum(x_ref[...] * wd_ref[...][:, None, :], axis=2)

    @pl.when(j == NSTEP3 - 1)
    def _():
        _head(racc_ref, z_ref, w1_ref, b1_ref, g_ref, bb_ref, wh1_ref,
              bh1_ref, wh2_ref, bh2_ref, out_ref)


def _head(r_ref, z_ref, w1_ref, b1_ref, g_ref, bb_ref, wh1_ref, bh1_ref,
          wh2_ref, bh2_ref, out_ref):
    r = r_ref[...] / z_ref[...][:, 0:1]
    out = lax.dot_general(r, w1_ref[...], (((1,), (1,)), ((), ())),
                          precision=lax.Precision.HIGHEST,
                          preferred_element_type=jnp.float32) + b1_ref[...]
    mu = jnp.mean(out, axis=1, keepdims=True)
    dlt = out - mu
    var = jnp.mean(dlt * dlt, axis=1, keepdims=True)
    outn = dlt * lax.rsqrt(var + 1e-5) * g_ref[...] + bb_ref[...]
    h1 = lax.dot_general(outn, wh1_ref[...], (((1,), (1,)), ((), ())),
                         precision=lax.Precision.HIGHEST,
                         preferred_element_type=jnp.float32) + bh1_ref[...]
    h1 = 0.5 * h1 * (1.0 + lax.erf(h1 * (2.0 ** -0.5)))
    out_ref[...] = lax.dot_general(h1, wh2_ref[...], (((1,), (1,)), ((), ())),
                                   precision=lax.Precision.HIGHEST,
                                   preferred_element_type=jnp.float32) + bh2_ref[...]


@jax.jit
def kernel(x, W1, b1, Ws, bs, ln_g, ln_b, Wh1, bh1, Wh2, bh2):
    v = jnp.einsum("pl,p->l", W1, Ws[0])

    scores = pl.pallas_call(
        _scores_body,
        grid=(B // BBLK, D // DBLK),
        in_specs=[
            pl.BlockSpec(memory_space=pltpu.SMEM),
            pl.BlockSpec((BBLK, L, DBLK), lambda i, j: (i, 0, j)),
        ],
        out_specs=pl.BlockSpec((BBLK, DBLK), lambda i, j: (i, j)),
        out_shape=jax.ShapeDtypeStruct((B, D), jnp.float32),
    )(v, x)

    w_field, z = _sc_topk(scores)

    logits = pl.pallas_call(
        _reduce_head_body,
        grid=(NSTEP3,),
        in_specs=[
            pl.BlockSpec((B, L, DBLK3), lambda j: (0, 0, j)),
            pl.BlockSpec((B, DBLK3), lambda j: (0, j)),
            pl.BlockSpec((B, LANES), lambda j: (0, 0)),
            pl.BlockSpec((P, L), lambda j: (0, 0)),
            pl.BlockSpec((1, P), lambda j: (0, 0)),
            pl.BlockSpec((1, P), lambda j: (0, 0)),
            pl.BlockSpec((1, P), lambda j: (0, 0)),
            pl.BlockSpec((128, P), lambda j: (0, 0)),
            pl.BlockSpec((1, 128), lambda j: (0, 0)),
            pl.BlockSpec((2, 128), lambda j: (0, 0)),
            pl.BlockSpec((1, 2), lambda j: (0, 0)),
        ],
        out_specs=pl.BlockSpec((B, 2), lambda j: (0, 0)),
        out_shape=jax.ShapeDtypeStruct((B, 2), jnp.float32),
        scratch_shapes=[pltpu.VMEM((B, L), jnp.float32)],
    )(x, w_field, z, W1, b1.reshape(1, P), ln_g.reshape(1, P),
      ln_b.reshape(1, P), Wh1, bh1.reshape(1, 128), Wh2, bh2.reshape(1, 2))
    return logits
